# baseline (device time: 10180 ns/iter reference)
import jax
import jax.numpy as jnp
from jax import lax
from jax.experimental import pallas as pl
from jax.experimental.pallas import tpu as pltpu

N_DEV = 4
N_CHUNK = 2


def kernel(x):
    m, n = x.shape
    cm = m // N_CHUNK

    def body(x_ref, out_ref, stats_ref, send_sems, recv_sems):
        my = lax.axis_index("i")

        barrier_sem = pltpu.get_barrier_semaphore()
        for r in range(1, N_DEV):
            pl.semaphore_signal(
                barrier_sem, inc=1,
                device_id=((my + r) % N_DEV,),
                device_id_type=pl.DeviceIdType.MESH,
            )

        rdmas = [[None] * N_DEV for _ in range(N_CHUNK)]

        for c in range(N_CHUNK):
            rows = pl.ds(c * cm, cm)
            xv = x_ref[rows, :]
            lm = jnp.max(xv, axis=1, keepdims=True)
            e = jnp.exp(xv - lm)
            out_ref[rows, :] = e
            ls = jnp.sum(e, axis=1, keepdims=True)
            stats_ref[c, 0, :, :] = jnp.transpose(
                jnp.concatenate([lm, ls], axis=1), (1, 0)
            )
            if c == 0:
                pl.semaphore_wait(barrier_sem, N_DEV - 1)
            for r in range(1, N_DEV):
                rdma = pltpu.make_async_remote_copy(
                    src_ref=stats_ref.at[c, 0],
                    dst_ref=stats_ref.at[c, N_DEV - r],
                    send_sem=send_sems.at[c, r],
                    recv_sem=recv_sems.at[c, N_DEV - r],
                    device_id=((my + r) % N_DEV,),
                    device_id_type=pl.DeviceIdType.MESH,
                )
                rdma.start()
                rdmas[c][r] = rdma

        for c in range(N_CHUNK):
            for r in range(1, N_DEV):
                rdmas[c][r].wait()
            all_st = stats_ref[c, :, :, :]
            m_all = all_st[:, 0:1, :]
            s_all = all_st[:, 1:2, :]
            gm = jnp.max(m_all, axis=0)
            gs = jnp.sum(s_all * jnp.exp(m_all - gm), axis=0)
            scale_row = jnp.exp(all_st[0, 0:1, :] - gm) / gs
            rows = pl.ds(c * cm, cm)
            out_ref[rows, :] = out_ref[rows, :] * jnp.transpose(scale_row, (1, 0))

    return pl.pallas_call(
        body,
        out_shape=jax.ShapeDtypeStruct((m, n), x.dtype),
        in_specs=[pl.BlockSpec(memory_space=pltpu.VMEM)],
        out_specs=pl.BlockSpec(memory_space=pltpu.VMEM),
        scratch_shapes=[
            pltpu.VMEM((N_CHUNK, N_DEV, 2, cm), jnp.float32),
            pltpu.SemaphoreType.DMA((N_CHUNK, N_DEV)),
            pltpu.SemaphoreType.DMA((N_CHUNK, N_DEV)),
        ],
        compiler_params=pltpu.CompilerParams(collective_id=0),
    )(x)
